# baseline (speedup 1.0000x reference)
import functools

import jax
import jax.numpy as jnp
from jax import lax
from jax.experimental import pallas as pl
from jax.experimental.pallas import tpu as pltpu, tpu_sc as plsc

BATCH = 16384
FACTORS = 32

_info = plsc.get_sparse_core_info()
_NC, _NS, _L = _info.num_cores, _info.num_subcores, _info.num_lanes
_HALF = BATCH // _NC
_FPT = FACTORS // _NS

_mesh = plsc.VectorSubcoreMesh(core_axis_name="c", subcore_axis_name="s")


@functools.partial(
    pl.kernel,
    mesh=_mesh,
    out_type=jax.ShapeDtypeStruct((BATCH,), jnp.float32),
    compiler_params=pltpu.CompilerParams(needs_layout_passes=False,
                                         use_tc_tiling_on_sc=False),
    scratch_types=[
        pltpu.VMEM((_HALF,), jnp.int32),
        pltpu.VMEM((_HALF,), jnp.int32),
        pltpu.VMEM((_HALF,), jnp.float32),
        pltpu.VMEM((_HALF,), jnp.float32),
        pltpu.VMEM((_HALF,), jnp.float32),
        pltpu.VMEM((_HALF,), jnp.float32),
        pltpu.VMEM((_HALF,), jnp.float32),
        pltpu.VMEM((_HALF,), jnp.float32),
        pltpu.VMEM((_L,), jnp.float32),
        pltpu.VMEM((_NS, _HALF // _NS), jnp.float32),
        pltpu.VMEM((_HALF // _NS,), jnp.float32),
        pltpu.VMEM_SHARED((_NS, _HALF), jnp.float32),
        pltpu.SemaphoreType.DMA,
    ],
)
def _mf_kernel(user_hbm, item_hbm, ut_hbm, it_hbm, bu_hbm, bi_hbm, gb_hbm,
               out_hbm,
               uidx_v, iidx_v, u0_v, u1_v, i0_v, i1_v, part_v, bv_v, gb_v,
               red_v, res_v, shared, sem):
    sc = lax.axis_index("c")
    t = lax.axis_index("s")
    base = sc * _HALF
    f0 = t * _FPT
    f1 = f0 + 1

    pltpu.sync_copy(user_hbm.at[pl.ds(base, _HALF)], uidx_v)
    pltpu.sync_copy(item_hbm.at[pl.ds(base, _HALF)], iidx_v)

    cps = [
        pltpu.async_copy(ut_hbm.at[f0].at[uidx_v], u0_v, sem),
        pltpu.async_copy(ut_hbm.at[f1].at[uidx_v], u1_v, sem),
        pltpu.async_copy(it_hbm.at[f0].at[iidx_v], i0_v, sem),
        pltpu.async_copy(it_hbm.at[f1].at[iidx_v], i1_v, sem),
    ]
    @pl.when(t == 0)
    def _():
        pltpu.async_copy(bu_hbm.at[uidx_v], bv_v, sem).wait()

    @pl.when(t == 1)
    def _():
        pltpu.async_copy(bi_hbm.at[iidx_v], bv_v, sem).wait()
    pltpu.sync_copy(gb_hbm, gb_v)
    for cp in cps:
        cp.wait()

    gb = gb_v[...]

    def prod_body(g, carry):
        o = g * _L
        p = (u0_v[pl.ds(o, _L)] * i0_v[pl.ds(o, _L)]
             + u1_v[pl.ds(o, _L)] * i1_v[pl.ds(o, _L)])
        part_v[pl.ds(o, _L)] = p
        return carry

    lax.fori_loop(0, _HALF // _L, prod_body, 0)

    def bias0_body(g, carry):
        o = g * _L
        part_v[pl.ds(o, _L)] = part_v[pl.ds(o, _L)] + bv_v[pl.ds(o, _L)] + gb
        return carry

    def bias1_body(g, carry):
        o = g * _L
        part_v[pl.ds(o, _L)] = part_v[pl.ds(o, _L)] + bv_v[pl.ds(o, _L)]
        return carry

    @pl.when(t == 0)
    def _():
        lax.fori_loop(0, _HALF // _L, bias0_body, 0)

    @pl.when(t == 1)
    def _():
        lax.fori_loop(0, _HALF // _L, bias1_body, 0)

    pltpu.sync_copy(part_v, shared.at[t])
    plsc.subcore_barrier()

    stripe = _HALF // _NS
    off = t * stripe
    red_cps = [
        pltpu.async_copy(shared.at[r, pl.ds(off, stripe)], red_v.at[r], sem)
        for r in range(_NS)
    ]
    for cp in red_cps:
        cp.wait()

    def red_body(g, carry):
        o = g * _L
        acc = red_v[0, pl.ds(o, _L)]
        for r in range(1, _NS):
            acc = acc + red_v[r, pl.ds(o, _L)]
        res_v[pl.ds(o, _L)] = acc
        return carry

    lax.fori_loop(0, stripe // _L, red_body, 0)

    pltpu.sync_copy(res_v, out_hbm.at[pl.ds(base + off, stripe)])


def kernel(user, item, U_MF, I_MF, B_U, B_I, GB):
    ut = U_MF.T
    it = I_MF.T
    bu = B_U.reshape(-1)
    bi = B_I.reshape(-1)
    gb_vec = jnp.broadcast_to(GB.astype(jnp.float32).reshape(1), (_L,))
    return _mf_kernel(user.astype(jnp.int32), item.astype(jnp.int32),
                      ut, it, bu, bi, gb_vec)


# speedup vs baseline: 15.4553x; 15.4553x over previous
import functools

import jax
import jax.numpy as jnp
from jax import lax
from jax.experimental import pallas as pl
from jax.experimental.pallas import tpu as pltpu, tpu_sc as plsc

BATCH = 16384
FACTORS = 32

_info = plsc.get_sparse_core_info()
_NC, _NS, _L = _info.num_cores, _info.num_subcores, _info.num_lanes
_NW = _NC * _NS
_BPW = BATCH // _NW
_GROUPS = _BPW // _L
_NSLOT = 4

_mesh = plsc.VectorSubcoreMesh(core_axis_name="c", subcore_axis_name="s")


@functools.partial(
    pl.kernel,
    mesh=_mesh,
    out_type=jax.ShapeDtypeStruct((BATCH,), jnp.float32),
    compiler_params=pltpu.CompilerParams(needs_layout_passes=False),
    scratch_types=(
        [
            pltpu.VMEM((_BPW,), jnp.int32),
            pltpu.VMEM((_BPW,), jnp.int32),
            pltpu.VMEM((_BPW,), jnp.float32),
            pltpu.VMEM((_BPW,), jnp.float32),
            pltpu.VMEM((_L,), jnp.float32),
            pltpu.VMEM((_BPW,), jnp.float32),
        ]
        + [pltpu.VMEM((FACTORS, 128), jnp.float32) for _ in range(2 * _NSLOT)]
        + [pltpu.SemaphoreType.DMA for _ in range(2 * _NSLOT)]
        + [pltpu.SemaphoreType.DMA]
    ),
)
def _mf_kernel(user_hbm, item_hbm, ut_hbm, it_hbm, bu_hbm, bi_hbm, gb_hbm,
               out_hbm,
               uidx_v, iidx_v, bu_v, bi_v, gb_v, out_v,
               *bufs_and_sems):
    ubufs = bufs_and_sems[:_NSLOT]
    ibufs = bufs_and_sems[_NSLOT:2 * _NSLOT]
    usems = bufs_and_sems[2 * _NSLOT:3 * _NSLOT]
    isems = bufs_and_sems[3 * _NSLOT:4 * _NSLOT]
    bsem = bufs_and_sems[4 * _NSLOT]

    wid = lax.axis_index("s") * _NC + lax.axis_index("c")
    base = wid * _BPW

    pltpu.sync_copy(user_hbm.at[pl.ds(base, _BPW)], uidx_v)
    pltpu.sync_copy(item_hbm.at[pl.ds(base, _BPW)], iidx_v)
    pltpu.sync_copy(gb_hbm, gb_v)

    bcp_u = pltpu.async_copy(bu_hbm.at[uidx_v], bu_v, bsem)
    bcp_i = pltpu.async_copy(bi_hbm.at[iidx_v], bi_v, bsem)

    lanes = lax.iota(jnp.int32, _L)
    last_lane = lanes == (_L - 1)
    lanes_hi = lanes + _L

    def group_body(g, carry):
        row0 = g * _L
        uvec = uidx_v[pl.ds(row0, _L)]
        ivec = iidx_v[pl.ds(row0, _L)]

        ucps = [None] * _L
        icps = [None] * _L

        def issue(jj):
            s = jj % _NSLOT
            ustart = pl.multiple_of((uvec[jj] >> 7) * 128, 128)
            istart = pl.multiple_of((ivec[jj] >> 7) * 128, 128)
            ucps[jj] = pltpu.async_copy(
                ut_hbm.at[:, pl.ds(ustart, 128)], ubufs[s], usems[s])
            icps[jj] = pltpu.async_copy(
                it_hbm.at[:, pl.ds(istart, 128)], ibufs[s], isems[s])

        def extract(jj):
            s = jj % _NSLOT
            ucps[jj].wait()
            icps[jj].wait()
            ucol = jnp.broadcast_to(uvec[jj] & 127, (_L,))
            icol = jnp.broadcast_to(ivec[jj] & 127, (_L,))
            u0 = plsc.load_gather(ubufs[s], [lanes, ucol])
            u1 = plsc.load_gather(ubufs[s], [lanes_hi, ucol])
            v0 = plsc.load_gather(ibufs[s], [lanes, icol])
            v1 = plsc.load_gather(ibufs[s], [lanes_hi, icol])
            ssum = plsc.cumsum(u0 * v0 + u1 * v1)
            plsc.store_scatter(out_v,
                               [jnp.full((_L,), row0 + jj, jnp.int32)],
                               ssum, mask=last_lane)

        for jj in range(_L + _NSLOT):
            if jj >= _NSLOT:
                extract(jj - _NSLOT)
            if jj < _L:
                issue(jj)
        return carry

    lax.fori_loop(0, _GROUPS, group_body, 0)

    bcp_u.wait()
    bcp_i.wait()
    gb = gb_v[...]

    def bias_body(g, carry):
        o = g * _L
        out_v[pl.ds(o, _L)] = (out_v[pl.ds(o, _L)] + bu_v[pl.ds(o, _L)]
                               + bi_v[pl.ds(o, _L)] + gb)
        return carry

    lax.fori_loop(0, _GROUPS, bias_body, 0)

    pltpu.sync_copy(out_v, out_hbm.at[pl.ds(base, _BPW)])


def kernel(user, item, U_MF, I_MF, B_U, B_I, GB):
    ut = U_MF.T
    it = I_MF.T
    bu = B_U.reshape(-1)
    bi = B_I.reshape(-1)
    gb_vec = jnp.broadcast_to(GB.astype(jnp.float32).reshape(1), (_L,))
    return _mf_kernel(user.astype(jnp.int32), item.astype(jnp.int32),
                      ut, it, bu, bi, gb_vec)


# speedup vs baseline: 15.5596x; 1.0067x over previous
import functools

import jax
import jax.numpy as jnp
from jax import lax
from jax.experimental import pallas as pl
from jax.experimental.pallas import tpu as pltpu, tpu_sc as plsc

BATCH = 16384
FACTORS = 32

_info = plsc.get_sparse_core_info()
_NC, _NS, _L = _info.num_cores, _info.num_subcores, _info.num_lanes
_NW = _NC * _NS
_BPW = BATCH // _NW
_GROUPS = _BPW // _L
_NSLOT = 8

_mesh = plsc.VectorSubcoreMesh(core_axis_name="c", subcore_axis_name="s")


@functools.partial(
    pl.kernel,
    mesh=_mesh,
    out_type=jax.ShapeDtypeStruct((BATCH,), jnp.float32),
    compiler_params=pltpu.CompilerParams(needs_layout_passes=False),
    scratch_types=(
        [
            pltpu.VMEM((_BPW,), jnp.int32),
            pltpu.VMEM((_BPW,), jnp.int32),
            pltpu.VMEM((_BPW,), jnp.float32),
            pltpu.VMEM((_BPW,), jnp.float32),
            pltpu.VMEM((_L,), jnp.float32),
            pltpu.VMEM((_BPW,), jnp.float32),
        ]
        + [pltpu.VMEM((FACTORS, 128), jnp.float32) for _ in range(2 * _NSLOT)]
        + [pltpu.SemaphoreType.DMA for _ in range(2 * _NSLOT)]
        + [pltpu.SemaphoreType.DMA]
    ),
)
def _mf_kernel(user_hbm, item_hbm, ut_hbm, it_hbm, bu_hbm, bi_hbm, gb_hbm,
               out_hbm,
               uidx_v, iidx_v, bu_v, bi_v, gb_v, out_v,
               *bufs_and_sems):
    ubufs = bufs_and_sems[:_NSLOT]
    ibufs = bufs_and_sems[_NSLOT:2 * _NSLOT]
    usems = bufs_and_sems[2 * _NSLOT:3 * _NSLOT]
    isems = bufs_and_sems[3 * _NSLOT:4 * _NSLOT]
    bsem = bufs_and_sems[4 * _NSLOT]

    wid = lax.axis_index("s") * _NC + lax.axis_index("c")
    base = wid * _BPW

    pltpu.sync_copy(user_hbm.at[pl.ds(base, _BPW)], uidx_v)
    pltpu.sync_copy(item_hbm.at[pl.ds(base, _BPW)], iidx_v)
    pltpu.sync_copy(gb_hbm, gb_v)

    bcp_u = pltpu.async_copy(bu_hbm.at[uidx_v], bu_v, bsem)
    bcp_i = pltpu.async_copy(bi_hbm.at[iidx_v], bi_v, bsem)

    lanes = lax.iota(jnp.int32, _L)
    last_lane = lanes == (_L - 1)
    lanes_hi = lanes + _L

    def group_body(g, carry):
        row0 = g * _L
        uvec = uidx_v[pl.ds(row0, _L)]
        ivec = iidx_v[pl.ds(row0, _L)]

        ucps = [None] * _L
        icps = [None] * _L

        def issue(jj):
            s = jj % _NSLOT
            ustart = pl.multiple_of((uvec[jj] >> 7) * 128, 128)
            istart = pl.multiple_of((ivec[jj] >> 7) * 128, 128)
            ucps[jj] = pltpu.async_copy(
                ut_hbm.at[:, pl.ds(ustart, 128)], ubufs[s], usems[s])
            icps[jj] = pltpu.async_copy(
                it_hbm.at[:, pl.ds(istart, 128)], ibufs[s], isems[s])

        def extract(jj):
            s = jj % _NSLOT
            ucps[jj].wait()
            icps[jj].wait()
            ucol = jnp.broadcast_to(uvec[jj] & 127, (_L,))
            icol = jnp.broadcast_to(ivec[jj] & 127, (_L,))
            u0 = plsc.load_gather(ubufs[s], [lanes, ucol])
            u1 = plsc.load_gather(ubufs[s], [lanes_hi, ucol])
            v0 = plsc.load_gather(ibufs[s], [lanes, icol])
            v1 = plsc.load_gather(ibufs[s], [lanes_hi, icol])
            ssum = plsc.cumsum(u0 * v0 + u1 * v1)
            plsc.store_scatter(out_v,
                               [jnp.full((_L,), row0 + jj, jnp.int32)],
                               ssum, mask=last_lane)

        for jj in range(_L + _NSLOT):
            if jj >= _NSLOT:
                extract(jj - _NSLOT)
            if jj < _L:
                issue(jj)
        return carry

    lax.fori_loop(0, _GROUPS, group_body, 0)

    bcp_u.wait()
    bcp_i.wait()
    gb = gb_v[...]

    def bias_body(g, carry):
        o = g * _L
        out_v[pl.ds(o, _L)] = (out_v[pl.ds(o, _L)] + bu_v[pl.ds(o, _L)]
                               + bi_v[pl.ds(o, _L)] + gb)
        return carry

    lax.fori_loop(0, _GROUPS, bias_body, 0)

    pltpu.sync_copy(out_v, out_hbm.at[pl.ds(base, _BPW)])


def kernel(user, item, U_MF, I_MF, B_U, B_I, GB):
    ut = U_MF.T
    it = I_MF.T
    bu = B_U.reshape(-1)
    bi = B_I.reshape(-1)
    gb_vec = jnp.broadcast_to(GB.astype(jnp.float32).reshape(1), (_L,))
    return _mf_kernel(user.astype(jnp.int32), item.astype(jnp.int32),
                      ut, it, bu, bi, gb_vec)
